# baseline (speedup 1.0000x reference)
import functools

import jax
import jax.numpy as jnp
from jax import lax
from jax.experimental import pallas as pl
from jax.experimental.pallas import tpu as pltpu
from jax.experimental.pallas import tpu_sc as plsc

B, C, H, W = 16, 80, 128, 128
HW = H * W
N = B * HW
P = 2048
ALPHA = 0.25

TOT_ROWS = (B * C * H * W) // 128
SC_ROWS = 24576
TC_ROWS = TOT_ROWS - SC_ROWS
RB = 8192
NBLK = TC_ROWS // RB
NROW = 12

LN2 = 0.6931471805599453
LOG1P_COEF = (6.0578476e-09, 9.9999881e-01, -4.9995893e-01, 3.3278534e-01,
              -2.4618968e-01, 1.8421386e-01, -1.2447195e-01, 6.5735526e-02,
              -2.2628007e-02, 3.6622423e-03)

SC_CHUNK = 8192
SC_UNROLL = 4


def _sc_info():
    try:
        info = plsc.get_sparse_core_info()
        return info.num_cores, info.num_subcores
    except Exception:
        return 2, 16


def _f0_vreg(x):
    u = jnp.exp(-x)
    w = 1.0 + u
    bb = lax.bitcast_convert_type(w, jnp.int32)
    e = lax.shift_right_logical(bb, 23) - 127
    mb = lax.bitwise_or(lax.bitwise_and(bb, 0x007FFFFF), 0x3F800000)
    tm = lax.bitcast_convert_type(mb, jnp.float32) - 1.0
    pol = jnp.full((16,), LOG1P_COEF[-1], jnp.float32)
    for cco in LOG1P_COEF[-2::-1]:
        pol = pol * tm + cco
    lw = e.astype(jnp.float32) * LN2 + pol
    return (x + lw) / (w * w)


def _make_sc_kernel():
    NC, NS = _sc_info()
    NW = NC * NS
    PB = P // NW
    share = SC_ROWS * 128 // NW
    n_chunks = share // SC_CHUNK
    zc = (P - 16 * NW) // NW

    mesh = plsc.VectorSubcoreMesh(core_axis_name="c", subcore_axis_name="s")

    @functools.partial(
        pl.kernel,
        mesh=mesh,
        out_type=jax.ShapeDtypeStruct((NROW, P), jnp.float32),
        scratch_types=[
            pltpu.VMEM((PB,), jnp.int32),
            pltpu.VMEM((PB,), jnp.int32),
            pltpu.VMEM((PB,), jnp.int32),
            pltpu.VMEM((4 * PB,), jnp.int32),
            pltpu.VMEM((PB,), jnp.float32),
            pltpu.VMEM((4 * PB,), jnp.float32),
            pltpu.VMEM((SC_CHUNK,), jnp.float32),
            pltpu.VMEM((SC_CHUNK,), jnp.float32),
            pltpu.SemaphoreType.DMA,
            pltpu.SemaphoreType.DMA,
            pltpu.SemaphoreType.DMA,
        ],
    )
    def sc_kernel(pos_hbm, lab_hbm, logit_hbm, box_hbm, ctr_hbm, tgt_hbm,
                  out_hbm, pos_v, lab_v, idx_v, idx4_v, f_v, f4_v,
                  cbuf_a, cbuf_b, sem, sem_a, sem_b):
        wid = lax.axis_index("s") * NC + lax.axis_index("c")
        base = wid * PB
        pltpu.sync_copy(pos_hbm.at[pl.ds(base, PB)], pos_v)
        pltpu.async_copy(lab_hbm.at[pos_v], lab_v, sem).wait()

        for j in range(PB // 16):
            p = pos_v[pl.ds(j * 16, 16)]
            f_v[pl.ds(j * 16, 16)] = p.astype(jnp.float32)
        pltpu.sync_copy(f_v, out_hbm.at[10, pl.ds(base, PB)])

        pltpu.async_copy(ctr_hbm.at[pos_v], f_v, sem).wait()
        pltpu.sync_copy(f_v, out_hbm.at[1, pl.ds(base, PB)])

        for j in range(PB // 16):
            p = pos_v[pl.ds(j * 16, 16)]
            lb = lab_v[pl.ds(j * 16, 16)]
            b_ = lax.shift_right_logical(p, 14)
            hw = lax.bitwise_and(p, HW - 1)
            idx_v[pl.ds(j * 16, 16)] = b_ * (C * HW) + lb * HW + hw
        pltpu.async_copy(logit_hbm.at[idx_v], f_v, sem).wait()
        pltpu.sync_copy(f_v, out_hbm.at[0, pl.ds(base, PB)])

        for k in range(4):
            for j in range(PB // 16):
                p = pos_v[pl.ds(j * 16, 16)]
                b_ = lax.shift_right_logical(p, 14)
                hw = lax.bitwise_and(p, HW - 1)
                idx4_v[pl.ds(k * PB + j * 16, 16)] = (b_ * 4 + k) * HW + hw
        pltpu.async_copy(box_hbm.at[idx4_v], f4_v, sem).wait()
        for k in range(4):
            pltpu.sync_copy(f4_v.at[pl.ds(k * PB, PB)],
                            out_hbm.at[2 + k, pl.ds(base, PB)])

        for k in range(4):
            for j in range(PB // 16):
                p = pos_v[pl.ds(j * 16, 16)]
                idx4_v[pl.ds(k * PB + j * 16, 16)] = p * 4 + k
        pltpu.async_copy(tgt_hbm.at[idx4_v], f4_v, sem).wait()
        for k in range(4):
            pltpu.sync_copy(f4_v.at[pl.ds(k * PB, PB)],
                            out_hbm.at[6 + k, pl.ds(base, PB)])

        start = TC_ROWS * 128 + wid * share
        bufs = (cbuf_a, cbuf_b)
        sems = (sem_a, sem_b)
        copies = [None, None]
        copies[0] = pltpu.async_copy(
            logit_hbm.at[pl.ds(start, SC_CHUNK)], cbuf_a, sem_a)

        for o in range(PB // 16):
            f_v[pl.ds(o * 16, 16)] = jnp.zeros((16,), jnp.float32)

        def vloop(buf):
            def body(i, carry):
                s = f_v[pl.ds(0, 16)]
                for o in range(SC_UNROLL):
                    xv = buf[pl.ds(i * (16 * SC_UNROLL) + o * 16, 16)]
                    s = s + _f0_vreg(xv)
                f_v[pl.ds(0, 16)] = s
                return carry
            lax.fori_loop(0, SC_CHUNK // (16 * SC_UNROLL), body, 0)

        for c in range(n_chunks):
            if c + 1 < n_chunks:
                copies[(c + 1) % 2] = pltpu.async_copy(
                    logit_hbm.at[pl.ds(start + (c + 1) * SC_CHUNK, SC_CHUNK)],
                    bufs[(c + 1) % 2], sems[(c + 1) % 2])
            copies[c % 2].wait()
            vloop(bufs[c % 2])
        pltpu.sync_copy(f_v, out_hbm.at[11, pl.ds(base, PB)])

    return sc_kernel


def _tc_sum_body(logit_ref, out_ref):
    i = pl.program_id(0)

    @pl.when(i == 0)
    def _init():
        out_ref[...] = jnp.zeros((1, 8, 128), jnp.float32)

    x = logit_ref[...]
    u = jnp.exp(-x)
    w = 1.0 + u
    r = 1.0 / w
    t = (x + jnp.log(w)) * r * r
    out_ref[...] += jnp.sum(t.reshape(RB // 8, 8, 128), axis=0)[None]


def _tc_final_body(gath_ref, part_ref, out_ref):
    g = gath_ref[...]
    xg = g[0]
    ug = jnp.exp(-xg)
    wg = 1.0 + ug
    rg = 1.0 / wg
    lwg = jnp.log(wg)
    f0 = (1.0 - ALPHA) * (xg + lwg) * rg * rg
    f1 = ALPHA * lwg * (ug * rg) * (ug * rg)

    pf = g[10]
    prev_in_row = jnp.concatenate(
        [jnp.full((16, 1), -1.0, jnp.float32), pf[:, :-1]], axis=1)
    row_carry = jnp.concatenate(
        [jnp.full((1, 1), -1.0, jnp.float32), pf[:-1, 127:128]], axis=0)
    ci = lax.broadcasted_iota(jnp.int32, (16, 128), 1)
    prev = jnp.where(ci == 0, row_carry, prev_in_row)
    corr = jnp.sum(jnp.where(pf != prev, f1 - f0, 0.0))
    dense = (1.0 - ALPHA) * (jnp.sum(part_ref[...]) + jnp.sum(g[11]))
    loss_cls = (dense + corr) / float(P)

    lp, tp, rp, bp = g[2], g[3], g[4], g[5]
    lt, tt, rt, bt = g[6], g[7], g[8], g[9]
    lr_min = jnp.minimum(lt, rt)
    lr_max = jnp.maximum(lt, rt)
    tb_min = jnp.minimum(tt, bt)
    tb_max = jnp.maximum(tt, bt)
    ctr_tgt = jnp.sqrt((lr_min / lr_max) * (tb_min / tb_max))
    denom = jnp.maximum(jnp.sum(ctr_tgt), 1e-6)
    target_area = (lt + rt) * (tt + bt)
    pred_area = (lp + rp) * (tp + bp)
    w_int = jnp.minimum(lp, lt) + jnp.minimum(rp, rt)
    h_int = jnp.minimum(tp, tt) + jnp.minimum(bp, bt)
    g_w = jnp.maximum(lp, lt) + jnp.maximum(rp, rt)
    g_h = jnp.maximum(tp, tt) + jnp.maximum(bp, bt)
    ac = g_w * g_h + 1e-7
    area_int = w_int * h_int
    area_union = target_area + pred_area - area_int
    ious = (area_int + 1.0) / (area_union + 1.0)
    gious = ious - (ac - area_union) / ac
    loss_box = jnp.sum((1.0 - gious) * ctr_tgt) / denom

    cp = g[1]
    bce = (jnp.maximum(cp, 0.0) - cp * ctr_tgt
           + jnp.log1p(jnp.exp(-jnp.abs(cp))))
    loss_ctr = jnp.sum(bce) / float(P)

    ri = lax.broadcasted_iota(jnp.int32, (8, 128), 0)
    cj = lax.broadcasted_iota(jnp.int32, (8, 128), 1)
    res = jnp.where(
        (ri == 0) & (cj == 0), loss_cls,
        jnp.where((ri == 0) & (cj == 1), loss_box,
                  jnp.where((ri == 0) & (cj == 2), loss_ctr, 0.0)))
    out_ref[...] = res


def _tc_call(g3, logits2d):
    partials = pl.pallas_call(
        _tc_sum_body,
        grid=(NBLK,),
        in_specs=[
            pl.BlockSpec((RB, 128), lambda i: (i, 0)),
        ],
        out_specs=pl.BlockSpec((1, 8, 128), lambda i: (0, 0, 0)),
        out_shape=jax.ShapeDtypeStruct((1, 8, 128), jnp.float32),
    )(logits2d)
    return pl.pallas_call(
        _tc_final_body,
        grid=(1,),
        in_specs=[
            pl.BlockSpec((NROW, 16, 128), lambda i: (0, 0, 0)),
            pl.BlockSpec((1, 8, 128), lambda i: (0, 0, 0)),
        ],
        out_specs=pl.BlockSpec((8, 128), lambda i: (0, 0)),
        out_shape=jax.ShapeDtypeStruct((8, 128), jnp.float32),
    )(g3, partials)


def kernel(logits, box2d_reg, centerness, labels, box2d_reg_targets, pos_inds):
    logits_flat = logits.reshape(-1)
    box_flat = box2d_reg.reshape(-1)
    ctr_flat = centerness.reshape(-1)
    tgt_flat = box2d_reg_targets.reshape(-1)
    pos = pos_inds.astype(jnp.int32)
    labs = labels.astype(jnp.int32)

    sc_kernel = _make_sc_kernel()
    g = sc_kernel(pos, labs, logits_flat, box_flat, ctr_flat, tgt_flat)
    g3 = g.reshape(NROW, 16, 128)
    out = _tc_call(g3, logits_flat.reshape(TOT_ROWS, 128))
    return out[0, :3]


# speedup vs baseline: 1.1257x; 1.1257x over previous
import functools

import jax
import jax.numpy as jnp
from jax import lax
from jax.experimental import pallas as pl
from jax.experimental.pallas import tpu as pltpu
from jax.experimental.pallas import tpu_sc as plsc

B, C, H, W = 16, 80, 128, 128
HW = H * W
N = B * HW
P = 2048
ALPHA, GAMMA = 0.25, 2.0

RB = 8192
TOT_ROWS = (B * C * H * W) // 128
HALF_ROWS = TOT_ROWS // 2
NBLK = HALF_ROWS // RB
NROW = 11


def _sc_info():
    try:
        info = plsc.get_sparse_core_info()
        return info.num_cores, info.num_subcores
    except Exception:
        return 2, 16


def _make_sc_gather():
    NC, NS = _sc_info()
    NW = NC * NS
    PB = P // NW

    mesh = plsc.VectorSubcoreMesh(core_axis_name="c", subcore_axis_name="s")

    @functools.partial(
        pl.kernel,
        mesh=mesh,
        out_type=jax.ShapeDtypeStruct((NROW, P), jnp.float32),
        scratch_types=[
            pltpu.VMEM((PB,), jnp.int32),
            pltpu.VMEM((PB,), jnp.int32),
            pltpu.VMEM((PB,), jnp.int32),
            pltpu.VMEM((4 * PB,), jnp.int32),
            pltpu.VMEM((PB,), jnp.float32),
            pltpu.VMEM((4 * PB,), jnp.float32),
            pltpu.SemaphoreType.DMA,
        ],
    )
    def sc_gather(pos_hbm, lab_hbm, logit_hbm, box_hbm, ctr_hbm, tgt_hbm,
                  out_hbm, pos_v, lab_v, idx_v, idx4_v, f_v, f4_v, sem):
        wid = lax.axis_index("s") * NC + lax.axis_index("c")
        base = wid * PB
        pltpu.sync_copy(pos_hbm.at[pl.ds(base, PB)], pos_v)
        pltpu.async_copy(lab_hbm.at[pos_v], lab_v, sem).wait()

        for j in range(PB // 16):
            p = pos_v[pl.ds(j * 16, 16)]
            f_v[pl.ds(j * 16, 16)] = p.astype(jnp.float32)
        pltpu.sync_copy(f_v, out_hbm.at[10, pl.ds(base, PB)])

        pltpu.async_copy(ctr_hbm.at[pos_v], f_v, sem).wait()
        pltpu.sync_copy(f_v, out_hbm.at[1, pl.ds(base, PB)])

        for j in range(PB // 16):
            p = pos_v[pl.ds(j * 16, 16)]
            lb = lab_v[pl.ds(j * 16, 16)]
            b_ = lax.shift_right_logical(p, 14)
            hw = lax.bitwise_and(p, HW - 1)
            idx_v[pl.ds(j * 16, 16)] = b_ * (C * HW) + lb * HW + hw
        pltpu.async_copy(logit_hbm.at[idx_v], f_v, sem).wait()
        pltpu.sync_copy(f_v, out_hbm.at[0, pl.ds(base, PB)])

        for k in range(4):
            for j in range(PB // 16):
                p = pos_v[pl.ds(j * 16, 16)]
                b_ = lax.shift_right_logical(p, 14)
                hw = lax.bitwise_and(p, HW - 1)
                idx4_v[pl.ds(k * PB + j * 16, 16)] = (b_ * 4 + k) * HW + hw
        pltpu.async_copy(box_hbm.at[idx4_v], f4_v, sem).wait()
        for k in range(4):
            pltpu.sync_copy(f4_v.at[pl.ds(k * PB, PB)],
                            out_hbm.at[2 + k, pl.ds(base, PB)])

        for k in range(4):
            for j in range(PB // 16):
                p = pos_v[pl.ds(j * 16, 16)]
                idx4_v[pl.ds(k * PB + j * 16, 16)] = p * 4 + k
        pltpu.async_copy(tgt_hbm.at[idx4_v], f4_v, sem).wait()
        for k in range(4):
            pltpu.sync_copy(f4_v.at[pl.ds(k * PB, PB)],
                            out_hbm.at[6 + k, pl.ds(base, PB)])

    return sc_gather


NBI = NBLK


def _f0sum(x):
    u = jnp.exp(-x)
    w = 1.0 + u
    r = 1.0 / w
    t = (x + jnp.log(w)) * r * r
    return jnp.sum(t.reshape(RB // 8, 8, 128), axis=0)[None]


def _tc_sum_body(gath_ref, la_ref, lb_ref, out_ref, acc_ref):
    i = pl.program_id(1)

    @pl.when(i == 0)
    def _init():
        acc_ref[...] = jnp.zeros((8, 128), jnp.float32)

    acc_ref[...] += _f0sum(la_ref[0])[0] + _f0sum(lb_ref[0])[0]

    @pl.when(i == NBI - 1)
    def _final():
        _tc_final(gath_ref, acc_ref, out_ref)


def _tc_final(gath_ref, acc_ref, out_ref):
    if True:
        g = gath_ref[...]
        xg = g[0]
        ug = jnp.exp(-xg)
        wg = 1.0 + ug
        rg = 1.0 / wg
        lwg = jnp.log(wg)
        f0 = (1.0 - ALPHA) * (xg + lwg) * rg * rg
        f1 = ALPHA * lwg * (ug * rg) * (ug * rg)

        pf = g[10]
        prev_in_row = jnp.concatenate(
            [jnp.full((16, 1), -1.0, jnp.float32), pf[:, :-1]], axis=1)
        row_carry = jnp.concatenate(
            [jnp.full((1, 1), -1.0, jnp.float32), pf[:-1, 127:128]], axis=0)
        ci = lax.broadcasted_iota(jnp.int32, (16, 128), 1)
        prev = jnp.where(ci == 0, row_carry, prev_in_row)
        corr = jnp.sum(jnp.where(pf != prev, f1 - f0, 0.0))
        dense = (1.0 - ALPHA) * jnp.sum(acc_ref[...])
        loss_cls = (dense + corr) / float(P)

        lp, tp, rp, bp = g[2], g[3], g[4], g[5]
        lt, tt, rt, bt = g[6], g[7], g[8], g[9]
        lr_min = jnp.minimum(lt, rt)
        lr_max = jnp.maximum(lt, rt)
        tb_min = jnp.minimum(tt, bt)
        tb_max = jnp.maximum(tt, bt)
        ctr_tgt = jnp.sqrt((lr_min / lr_max) * (tb_min / tb_max))
        denom = jnp.maximum(jnp.sum(ctr_tgt), 1e-6)
        target_area = (lt + rt) * (tt + bt)
        pred_area = (lp + rp) * (tp + bp)
        w_int = jnp.minimum(lp, lt) + jnp.minimum(rp, rt)
        h_int = jnp.minimum(tp, tt) + jnp.minimum(bp, bt)
        g_w = jnp.maximum(lp, lt) + jnp.maximum(rp, rt)
        g_h = jnp.maximum(tp, tt) + jnp.maximum(bp, bt)
        ac = g_w * g_h + 1e-7
        area_int = w_int * h_int
        area_union = target_area + pred_area - area_int
        ious = (area_int + 1.0) / (area_union + 1.0)
        gious = ious - (ac - area_union) / ac
        loss_box = jnp.sum((1.0 - gious) * ctr_tgt) / denom

        cp = g[1]
        bce = (jnp.maximum(cp, 0.0) - cp * ctr_tgt
               + jnp.log1p(jnp.exp(-jnp.abs(cp))))
        loss_ctr = jnp.sum(bce) / float(P)

        ri = lax.broadcasted_iota(jnp.int32, (8, 128), 0)
        cj = lax.broadcasted_iota(jnp.int32, (8, 128), 1)
        res = jnp.where(
            (ri == 0) & (cj == 0), loss_cls,
            jnp.where((ri == 0) & (cj == 1), loss_box,
                      jnp.where((ri == 0) & (cj == 2), loss_ctr, 0.0)))
        out_ref[...] = res


def _tc_call(g3, logits2d):
    l3 = logits2d.reshape(2, HALF_ROWS, 128)
    return pl.pallas_call(
        _tc_sum_body,
        grid=(1, NBI),
        in_specs=[
            pl.BlockSpec((NROW, 16, 128), lambda s, i: (0, 0, 0)),
            pl.BlockSpec((1, RB, 128), lambda s, i: (0, i, 0)),
            pl.BlockSpec((1, RB, 128), lambda s, i: (1, i, 0)),
        ],
        out_specs=pl.BlockSpec((8, 128), lambda s, i: (0, 0)),
        out_shape=jax.ShapeDtypeStruct((8, 128), jnp.float32),
        scratch_shapes=[pltpu.VMEM((8, 128), jnp.float32)],
    )(g3, l3, l3)


def kernel(logits, box2d_reg, centerness, labels, box2d_reg_targets, pos_inds):
    logits_flat = logits.reshape(-1)
    box_flat = box2d_reg.reshape(-1)
    ctr_flat = centerness.reshape(-1)
    tgt_flat = box2d_reg_targets.reshape(-1)
    pos = pos_inds.astype(jnp.int32)
    labs = labels.astype(jnp.int32)

    sc_gather = _make_sc_gather()
    g = sc_gather(pos, labs, logits_flat, box_flat, ctr_flat, tgt_flat)
    g3 = g.reshape(NROW, 16, 128)
    out = _tc_call(g3, logits_flat.reshape(TOT_ROWS, 128))
    return out[0, :3]


# speedup vs baseline: 1.1529x; 1.0241x over previous
import functools

import jax
import jax.numpy as jnp
from jax import lax
from jax.experimental import pallas as pl
from jax.experimental.pallas import tpu as pltpu
from jax.experimental.pallas import tpu_sc as plsc

B, C, H, W = 16, 80, 128, 128
HW = H * W
N = B * HW
P = 2048
ALPHA, GAMMA = 0.25, 2.0

RB = 8192
TOT_ROWS = (B * C * H * W) // 128
QUARTER_ROWS = TOT_ROWS // 4
NBLK = QUARTER_ROWS // RB
NROW = 11


def _sc_info():
    try:
        info = plsc.get_sparse_core_info()
        return info.num_cores, info.num_subcores
    except Exception:
        return 2, 16


def _make_sc_gather():
    NC, NS = _sc_info()
    NW = NC * NS
    PB = P // NW

    mesh = plsc.VectorSubcoreMesh(core_axis_name="c", subcore_axis_name="s")

    @functools.partial(
        pl.kernel,
        mesh=mesh,
        out_type=jax.ShapeDtypeStruct((NROW, P), jnp.float32),
        scratch_types=[
            pltpu.VMEM((PB,), jnp.int32),
            pltpu.VMEM((PB,), jnp.int32),
            pltpu.VMEM((PB,), jnp.int32),
            pltpu.VMEM((4 * PB,), jnp.int32),
            pltpu.VMEM((PB,), jnp.float32),
            pltpu.VMEM((4 * PB,), jnp.float32),
            pltpu.SemaphoreType.DMA,
        ],
    )
    def sc_gather(pos_hbm, lab_hbm, logit_hbm, box_hbm, ctr_hbm, tgt_hbm,
                  out_hbm, pos_v, lab_v, idx_v, idx4_v, f_v, f4_v, sem):
        wid = lax.axis_index("s") * NC + lax.axis_index("c")
        base = wid * PB
        pltpu.sync_copy(pos_hbm.at[pl.ds(base, PB)], pos_v)
        pltpu.async_copy(lab_hbm.at[pos_v], lab_v, sem).wait()

        for j in range(PB // 16):
            p = pos_v[pl.ds(j * 16, 16)]
            f_v[pl.ds(j * 16, 16)] = p.astype(jnp.float32)
        pltpu.sync_copy(f_v, out_hbm.at[10, pl.ds(base, PB)])

        pltpu.async_copy(ctr_hbm.at[pos_v], f_v, sem).wait()
        pltpu.sync_copy(f_v, out_hbm.at[1, pl.ds(base, PB)])

        for j in range(PB // 16):
            p = pos_v[pl.ds(j * 16, 16)]
            lb = lab_v[pl.ds(j * 16, 16)]
            b_ = lax.shift_right_logical(p, 14)
            hw = lax.bitwise_and(p, HW - 1)
            idx_v[pl.ds(j * 16, 16)] = b_ * (C * HW) + lb * HW + hw
        pltpu.async_copy(logit_hbm.at[idx_v], f_v, sem).wait()
        pltpu.sync_copy(f_v, out_hbm.at[0, pl.ds(base, PB)])

        for k in range(4):
            for j in range(PB // 16):
                p = pos_v[pl.ds(j * 16, 16)]
                b_ = lax.shift_right_logical(p, 14)
                hw = lax.bitwise_and(p, HW - 1)
                idx4_v[pl.ds(k * PB + j * 16, 16)] = (b_ * 4 + k) * HW + hw
        pltpu.async_copy(box_hbm.at[idx4_v], f4_v, sem).wait()
        for k in range(4):
            pltpu.sync_copy(f4_v.at[pl.ds(k * PB, PB)],
                            out_hbm.at[2 + k, pl.ds(base, PB)])

        for k in range(4):
            for j in range(PB // 16):
                p = pos_v[pl.ds(j * 16, 16)]
                idx4_v[pl.ds(k * PB + j * 16, 16)] = p * 4 + k
        pltpu.async_copy(tgt_hbm.at[idx4_v], f4_v, sem).wait()
        for k in range(4):
            pltpu.sync_copy(f4_v.at[pl.ds(k * PB, PB)],
                            out_hbm.at[6 + k, pl.ds(base, PB)])

    return sc_gather


NSPLIT = 1
NBI = NBLK


def _f0sum(x):
    u = jnp.exp(-x)
    w = 1.0 + u
    r = 1.0 / w
    t = (x + jnp.log(w)) * r * r
    return jnp.sum(t.reshape(RB // 8, 8, 128), axis=0)[None]


def _tc_sum_body(la_ref, lb_ref, lc_ref, ld_ref, out_ref):
    i = pl.program_id(1)

    @pl.when(i == 0)
    def _init():
        out_ref[...] = jnp.zeros((1, 8, 128), jnp.float32)

    out_ref[...] += ((_f0sum(la_ref[0]) + _f0sum(lb_ref[0]))
                     + (_f0sum(lc_ref[0]) + _f0sum(ld_ref[0])))


def _tc_final_body(gath_ref, part_ref, out_ref):
    if True:
        g = gath_ref[...]
        xg = g[0]
        ug = jnp.exp(-xg)
        wg = 1.0 + ug
        rg = 1.0 / wg
        lwg = jnp.log(wg)
        f0 = (1.0 - ALPHA) * (xg + lwg) * rg * rg
        f1 = ALPHA * lwg * (ug * rg) * (ug * rg)

        pf = g[10]
        prev_in_row = jnp.concatenate(
            [jnp.full((16, 1), -1.0, jnp.float32), pf[:, :-1]], axis=1)
        row_carry = jnp.concatenate(
            [jnp.full((1, 1), -1.0, jnp.float32), pf[:-1, 127:128]], axis=0)
        ci = lax.broadcasted_iota(jnp.int32, (16, 128), 1)
        prev = jnp.where(ci == 0, row_carry, prev_in_row)
        corr = jnp.sum(jnp.where(pf != prev, f1 - f0, 0.0))
        dense = (1.0 - ALPHA) * jnp.sum(part_ref[...])
        loss_cls = (dense + corr) / float(P)

        lp, tp, rp, bp = g[2], g[3], g[4], g[5]
        lt, tt, rt, bt = g[6], g[7], g[8], g[9]
        lr_min = jnp.minimum(lt, rt)
        lr_max = jnp.maximum(lt, rt)
        tb_min = jnp.minimum(tt, bt)
        tb_max = jnp.maximum(tt, bt)
        ctr_tgt = jnp.sqrt((lr_min / lr_max) * (tb_min / tb_max))
        denom = jnp.maximum(jnp.sum(ctr_tgt), 1e-6)
        target_area = (lt + rt) * (tt + bt)
        pred_area = (lp + rp) * (tp + bp)
        w_int = jnp.minimum(lp, lt) + jnp.minimum(rp, rt)
        h_int = jnp.minimum(tp, tt) + jnp.minimum(bp, bt)
        g_w = jnp.maximum(lp, lt) + jnp.maximum(rp, rt)
        g_h = jnp.maximum(tp, tt) + jnp.maximum(bp, bt)
        ac = g_w * g_h + 1e-7
        area_int = w_int * h_int
        area_union = target_area + pred_area - area_int
        ious = (area_int + 1.0) / (area_union + 1.0)
        gious = ious - (ac - area_union) / ac
        loss_box = jnp.sum((1.0 - gious) * ctr_tgt) / denom

        cp = g[1]
        bce = (jnp.maximum(cp, 0.0) - cp * ctr_tgt
               + jnp.log1p(jnp.exp(-jnp.abs(cp))))
        loss_ctr = jnp.sum(bce) / float(P)

        ri = lax.broadcasted_iota(jnp.int32, (8, 128), 0)
        cj = lax.broadcasted_iota(jnp.int32, (8, 128), 1)
        res = jnp.where(
            (ri == 0) & (cj == 0), loss_cls,
            jnp.where((ri == 0) & (cj == 1), loss_box,
                      jnp.where((ri == 0) & (cj == 2), loss_ctr, 0.0)))
        out_ref[...] = res


def _tc_call(g3, logits2d):
    l3 = logits2d.reshape(4, QUARTER_ROWS, 128)
    partials = pl.pallas_call(
        _tc_sum_body,
        grid=(NSPLIT, NBI),
        in_specs=[
            pl.BlockSpec((1, RB, 128), lambda s, i: (0, s * NBI + i, 0)),
            pl.BlockSpec((1, RB, 128), lambda s, i: (1, s * NBI + i, 0)),
            pl.BlockSpec((1, RB, 128), lambda s, i: (2, s * NBI + i, 0)),
            pl.BlockSpec((1, RB, 128), lambda s, i: (3, s * NBI + i, 0)),
        ],
        out_specs=pl.BlockSpec((1, 8, 128), lambda s, i: (s, 0, 0)),
        out_shape=jax.ShapeDtypeStruct((NSPLIT, 8, 128), jnp.float32),
        compiler_params=pltpu.CompilerParams(
            dimension_semantics=("parallel", "arbitrary")),
    )(l3, l3, l3, l3)
    return pl.pallas_call(
        _tc_final_body,
        grid=(1,),
        in_specs=[
            pl.BlockSpec((NROW, 16, 128), lambda i: (0, 0, 0)),
            pl.BlockSpec((NSPLIT, 8, 128), lambda i: (0, 0, 0)),
        ],
        out_specs=pl.BlockSpec((8, 128), lambda i: (0, 0)),
        out_shape=jax.ShapeDtypeStruct((8, 128), jnp.float32),
    )(g3, partials)


def kernel(logits, box2d_reg, centerness, labels, box2d_reg_targets, pos_inds):
    logits_flat = logits.reshape(-1)
    box_flat = box2d_reg.reshape(-1)
    ctr_flat = centerness.reshape(-1)
    tgt_flat = box2d_reg_targets.reshape(-1)
    pos = pos_inds.astype(jnp.int32)
    labs = labels.astype(jnp.int32)

    sc_gather = _make_sc_gather()
    g = sc_gather(pos, labs, logits_flat, box_flat, ctr_flat, tgt_flat)
    g3 = g.reshape(NROW, 16, 128)
    out = _tc_call(g3, logits_flat.reshape(TOT_ROWS, 128))
    return out[0, :3]


# speedup vs baseline: 1.1635x; 1.0092x over previous
import functools

import jax
import jax.numpy as jnp
from jax import lax
from jax.experimental import pallas as pl
from jax.experimental.pallas import tpu as pltpu
from jax.experimental.pallas import tpu_sc as plsc

B, C, H, W = 16, 80, 128, 128
HW = H * W
N = B * HW
P = 2048
ALPHA, GAMMA = 0.25, 2.0

RB = 8192
TOT_ROWS = (B * C * H * W) // 128
HALF_ROWS = TOT_ROWS // 2
NBLK = HALF_ROWS // RB
NROW = 11


def _sc_info():
    try:
        info = plsc.get_sparse_core_info()
        return info.num_cores, info.num_subcores
    except Exception:
        return 2, 16


def _make_sc_gather():
    NC, NS = _sc_info()
    NW = NC * NS
    PB = P // NW

    mesh = plsc.VectorSubcoreMesh(core_axis_name="c", subcore_axis_name="s")

    @functools.partial(
        pl.kernel,
        mesh=mesh,
        out_type=jax.ShapeDtypeStruct((NROW, P), jnp.float32),
        scratch_types=[
            pltpu.VMEM((PB,), jnp.int32),
            pltpu.VMEM((PB,), jnp.int32),
            pltpu.VMEM((PB,), jnp.int32),
            pltpu.VMEM((4 * PB,), jnp.int32),
            pltpu.VMEM((PB,), jnp.float32),
            pltpu.VMEM((4 * PB,), jnp.float32),
            pltpu.SemaphoreType.DMA,
        ],
    )
    def sc_gather(pos_hbm, lab_hbm, logit_hbm, box_hbm, ctr_hbm, tgt_hbm,
                  out_hbm, pos_v, lab_v, idx_v, idx4_v, f_v, f4_v, sem):
        wid = lax.axis_index("s") * NC + lax.axis_index("c")
        base = wid * PB
        pltpu.sync_copy(pos_hbm.at[pl.ds(base, PB)], pos_v)
        pltpu.async_copy(lab_hbm.at[pos_v], lab_v, sem).wait()

        for j in range(PB // 16):
            p = pos_v[pl.ds(j * 16, 16)]
            f_v[pl.ds(j * 16, 16)] = p.astype(jnp.float32)
        pltpu.sync_copy(f_v, out_hbm.at[10, pl.ds(base, PB)])

        pltpu.async_copy(ctr_hbm.at[pos_v], f_v, sem).wait()
        pltpu.sync_copy(f_v, out_hbm.at[1, pl.ds(base, PB)])

        for j in range(PB // 16):
            p = pos_v[pl.ds(j * 16, 16)]
            lb = lab_v[pl.ds(j * 16, 16)]
            b_ = lax.shift_right_logical(p, 14)
            hw = lax.bitwise_and(p, HW - 1)
            idx_v[pl.ds(j * 16, 16)] = b_ * (C * HW) + lb * HW + hw
        pltpu.async_copy(logit_hbm.at[idx_v], f_v, sem).wait()
        pltpu.sync_copy(f_v, out_hbm.at[0, pl.ds(base, PB)])

        for k in range(4):
            for j in range(PB // 16):
                p = pos_v[pl.ds(j * 16, 16)]
                b_ = lax.shift_right_logical(p, 14)
                hw = lax.bitwise_and(p, HW - 1)
                idx4_v[pl.ds(k * PB + j * 16, 16)] = (b_ * 4 + k) * HW + hw
        pltpu.async_copy(box_hbm.at[idx4_v], f4_v, sem).wait()
        for k in range(4):
            pltpu.sync_copy(f4_v.at[pl.ds(k * PB, PB)],
                            out_hbm.at[2 + k, pl.ds(base, PB)])

        for k in range(4):
            for j in range(PB // 16):
                p = pos_v[pl.ds(j * 16, 16)]
                idx4_v[pl.ds(k * PB + j * 16, 16)] = p * 4 + k
        pltpu.async_copy(tgt_hbm.at[idx4_v], f4_v, sem).wait()
        for k in range(4):
            pltpu.sync_copy(f4_v.at[pl.ds(k * PB, PB)],
                            out_hbm.at[6 + k, pl.ds(base, PB)])

    return sc_gather


NSPLIT = 2
NBI = NBLK // NSPLIT


def _f0sum(x):
    u = jnp.exp(-x)
    w = 1.0 + u
    r = 1.0 / w
    t = (x + jnp.log(w)) * r * r
    return jnp.sum(t.reshape(RB // 8, 8, 128), axis=0)[None]


def _tc_sum_body(la_ref, lb_ref, out_ref):
    i = pl.program_id(1)

    @pl.when(i == 0)
    def _init():
        out_ref[...] = jnp.zeros((1, 8, 128), jnp.float32)

    out_ref[...] += _f0sum(la_ref[0]) + _f0sum(lb_ref[0])


def _tc_final_body(gath_ref, part_ref, out_ref):
    if True:
        g = gath_ref[...]
        xg = g[0]
        ug = jnp.exp(-xg)
        wg = 1.0 + ug
        rg = 1.0 / wg
        lwg = jnp.log(wg)
        f0 = (1.0 - ALPHA) * (xg + lwg) * rg * rg
        f1 = ALPHA * lwg * (ug * rg) * (ug * rg)

        pf = g[10]
        prev_in_row = jnp.concatenate(
            [jnp.full((16, 1), -1.0, jnp.float32), pf[:, :-1]], axis=1)
        row_carry = jnp.concatenate(
            [jnp.full((1, 1), -1.0, jnp.float32), pf[:-1, 127:128]], axis=0)
        ci = lax.broadcasted_iota(jnp.int32, (16, 128), 1)
        prev = jnp.where(ci == 0, row_carry, prev_in_row)
        corr = jnp.sum(jnp.where(pf != prev, f1 - f0, 0.0))
        dense = (1.0 - ALPHA) * jnp.sum(part_ref[...])
        loss_cls = (dense + corr) / float(P)

        lp, tp, rp, bp = g[2], g[3], g[4], g[5]
        lt, tt, rt, bt = g[6], g[7], g[8], g[9]
        lr_min = jnp.minimum(lt, rt)
        lr_max = jnp.maximum(lt, rt)
        tb_min = jnp.minimum(tt, bt)
        tb_max = jnp.maximum(tt, bt)
        ctr_tgt = jnp.sqrt((lr_min / lr_max) * (tb_min / tb_max))
        denom = jnp.maximum(jnp.sum(ctr_tgt), 1e-6)
        target_area = (lt + rt) * (tt + bt)
        pred_area = (lp + rp) * (tp + bp)
        w_int = jnp.minimum(lp, lt) + jnp.minimum(rp, rt)
        h_int = jnp.minimum(tp, tt) + jnp.minimum(bp, bt)
        g_w = jnp.maximum(lp, lt) + jnp.maximum(rp, rt)
        g_h = jnp.maximum(tp, tt) + jnp.maximum(bp, bt)
        ac = g_w * g_h + 1e-7
        area_int = w_int * h_int
        area_union = target_area + pred_area - area_int
        ious = (area_int + 1.0) / (area_union + 1.0)
        gious = ious - (ac - area_union) / ac
        loss_box = jnp.sum((1.0 - gious) * ctr_tgt) / denom

        cp = g[1]
        bce = (jnp.maximum(cp, 0.0) - cp * ctr_tgt
               + jnp.log1p(jnp.exp(-jnp.abs(cp))))
        loss_ctr = jnp.sum(bce) / float(P)

        ri = lax.broadcasted_iota(jnp.int32, (8, 128), 0)
        cj = lax.broadcasted_iota(jnp.int32, (8, 128), 1)
        res = jnp.where(
            (ri == 0) & (cj == 0), loss_cls,
            jnp.where((ri == 0) & (cj == 1), loss_box,
                      jnp.where((ri == 0) & (cj == 2), loss_ctr, 0.0)))
        out_ref[...] = res


def _tc_call(g3, logits2d):
    l3 = logits2d.reshape(2, HALF_ROWS, 128)
    partials = pl.pallas_call(
        _tc_sum_body,
        grid=(NSPLIT, NBI),
        in_specs=[
            pl.BlockSpec((1, RB, 128), lambda s, i: (0, s * NBI + i, 0)),
            pl.BlockSpec((1, RB, 128), lambda s, i: (1, s * NBI + i, 0)),
        ],
        out_specs=pl.BlockSpec((1, 8, 128), lambda s, i: (s, 0, 0)),
        out_shape=jax.ShapeDtypeStruct((NSPLIT, 8, 128), jnp.float32),
        compiler_params=pltpu.CompilerParams(
            dimension_semantics=("parallel", "arbitrary")),
    )(l3, l3)
    return pl.pallas_call(
        _tc_final_body,
        grid=(1,),
        in_specs=[
            pl.BlockSpec((NROW, 16, 128), lambda i: (0, 0, 0)),
            pl.BlockSpec((NSPLIT, 8, 128), lambda i: (0, 0, 0)),
        ],
        out_specs=pl.BlockSpec((8, 128), lambda i: (0, 0)),
        out_shape=jax.ShapeDtypeStruct((8, 128), jnp.float32),
    )(g3, partials)


def kernel(logits, box2d_reg, centerness, labels, box2d_reg_targets, pos_inds):
    logits_flat = logits.reshape(-1)
    box_flat = box2d_reg.reshape(-1)
    ctr_flat = centerness.reshape(-1)
    tgt_flat = box2d_reg_targets.reshape(-1)
    pos = pos_inds.astype(jnp.int32)
    labs = labels.astype(jnp.int32)

    sc_gather = _make_sc_gather()
    g = sc_gather(pos, labs, logits_flat, box_flat, ctr_flat, tgt_flat)
    g3 = g.reshape(NROW, 16, 128)
    out = _tc_call(g3, logits_flat.reshape(TOT_ROWS, 128))
    return out[0, :3]
